# Initial kernel scaffold; baseline (speedup 1.0000x reference)
import jax
import jax.numpy as jnp
from jax.experimental import pallas as pl


def kernel(x, pos, graph_features, batch, edge_index, edge_attr, params):
    raise NotImplementedError("write your pallas kernel here")


# speedup vs baseline: 2.1374x; 2.1374x over previous
import functools

import jax
import jax.numpy as jnp
from jax import lax
from jax.experimental import pallas as pl
from jax.experimental.pallas import tpu as pltpu
from jax.experimental.pallas import tpu_sc as plsc

F32 = jnp.float32

_N = 50000
_E = 800000
_HID = 64
_NG = 8
_NL = 3

_NTILE = 32
_K = 128
_NCHUNK = _E // _K
_ROWS_A = 3200
_ROWS_B = 2000


def _silu(u):
    return u / (1.0 + jnp.exp(-u))


def _prep_body(Wqs, bqs, Wks, bks, Wvs, bvs, Wa1s, ba1s, wa2s, ba2s,
               We2s, be2s, Wm1s, bm1s, gf, Wge, bge,
               WAs, WBs, WVs, WCs, cCs, WMs, cMs, cons, G):
    for l in range(_NL):
        Wq, bq = Wqs[l], bqs[l]
        Wk, bk = Wks[l], bks[l]
        Wv, bv = Wvs[l], bvs[l]
        Wa1 = Wa1s[l]
        Wa1_q, Wa1_k, Wa1_e = Wa1[0:64, :], Wa1[64:128, :], Wa1[128:192, :]
        Wm1 = Wm1s[l]
        Wm1_v, Wm1_e = Wm1[0:64, :], Wm1[64:128, :]
        We2, be2 = We2s[l], be2s[l]
        dot = lambda x, y: jnp.dot(x, y, preferred_element_type=F32)
        WAs[l] = dot(Wq, Wa1_q)
        WBs[l] = dot(Wk, Wa1_k)
        WVs[l] = dot(Wv, Wm1_v)
        WCs[l] = dot(We2, Wa1_e)
        cCs[l] = ba1s[l] + dot(bq, Wa1_q) + dot(bk, Wa1_k) + dot(be2, Wa1_e)
        WMs[l] = dot(We2, Wm1_e)
        cMs[l] = bm1s[l] + dot(bv, Wm1_v) + dot(be2, Wm1_e)
        w2 = wa2s[l].reshape(4, 16)
        b2 = jnp.full((4, 16), ba2s[l][0, 0], F32)
        cons[l] = jnp.concatenate([w2, b2], axis=0)
    G[...] = jnp.dot(gf[...], Wge[...], preferred_element_type=F32) + bge[...]


def _run_prep(params, graph_features):
    blocks = params['blocks']
    st = lambda key: jnp.stack([b[key][0] for b in blocks])
    stb = lambda key: jnp.stack([b[key][1].reshape(1, -1) for b in blocks])
    Wge, bge = params['graph_enc']
    outs = pl.pallas_call(
        _prep_body,
        out_shape=[
            jax.ShapeDtypeStruct((_NL, 64, 64), F32),
            jax.ShapeDtypeStruct((_NL, 64, 64), F32),
            jax.ShapeDtypeStruct((_NL, 64, 64), F32),
            jax.ShapeDtypeStruct((_NL, 64, 64), F32),
            jax.ShapeDtypeStruct((_NL, 1, 64), F32),
            jax.ShapeDtypeStruct((_NL, 64, 64), F32),
            jax.ShapeDtypeStruct((_NL, 1, 64), F32),
            jax.ShapeDtypeStruct((_NL, 8, 16), F32),
            jax.ShapeDtypeStruct((_NG, 64), F32),
        ],
    )(st('q'), stb('q'), st('k'), stb('k'), st('v'), stb('v'),
      st('att1'), stb('att1'), st('att2'), stb('att2'),
      st('edge2'), stb('edge2'), st('msg1'), stb('msg1'),
      graph_features, Wge, bge.reshape(1, -1))
    return outs


def _enc_body(x, bcol, Wne, bne, G, h):
    oh = (bcol[...] == lax.broadcasted_iota(jnp.int32, (x.shape[0], _NG), 1))
    h[...] = (jnp.dot(x[...], Wne[...], preferred_element_type=F32) + bne[...]
              + jnp.dot(oh.astype(F32), G[...], preferred_element_type=F32))


def _run_enc(x, batch, Wne, bne, G):
    nb = 50
    bs = _N // nb
    return pl.pallas_call(
        _enc_body,
        grid=(nb,),
        in_specs=[
            pl.BlockSpec((bs, 128), lambda i: (i, 0)),
            pl.BlockSpec((bs, 1), lambda i: (i, 0)),
            pl.BlockSpec((128, 64), lambda i: (0, 0)),
            pl.BlockSpec((1, 64), lambda i: (0, 0)),
            pl.BlockSpec((_NG, 64), lambda i: (0, 0)),
        ],
        out_specs=pl.BlockSpec((bs, 64), lambda i: (i, 0)),
        out_shape=jax.ShapeDtypeStruct((_N, 64), F32),
    )(x, batch.reshape(-1, 1), Wne, bne.reshape(1, -1), G)


def _edgepre_body(a, w1s, b1s, WCs, cCs, WMs, cMs, C0, C1, C2, M0, M1, M2):
    Couts = (C0, C1, C2)
    Mouts = (M0, M1, M2)
    for l in range(_NL):
        r = jnp.maximum(a[...] * w1s[l] + b1s[l], 0.0)
        Couts[l][...] = (jnp.dot(r, WCs[l], preferred_element_type=F32)
                         + cCs[l])
        M = jnp.dot(r, WMs[l], preferred_element_type=F32) + cMs[l]
        for qq in range(4):
            Mouts[l][qq] = M[:, qq * 16:(qq + 1) * 16]


def _run_edgepre(edge_attr, w1s, b1s, WCs, cCs, WMs, cMs):
    nb = 400
    bs = _E // nb
    cshape = jax.ShapeDtypeStruct((_E, 64), F32)
    mshape = jax.ShapeDtypeStruct((4, _E, 16), F32)
    full = lambda s: pl.BlockSpec(s, lambda i: tuple(0 for _ in s))
    return pl.pallas_call(
        _edgepre_body,
        grid=(nb,),
        in_specs=[
            pl.BlockSpec((bs, 1), lambda i: (i, 0)),
            full((_NL, 1, 64)), full((_NL, 1, 64)),
            full((_NL, 64, 64)), full((_NL, 1, 64)),
            full((_NL, 64, 64)), full((_NL, 1, 64)),
        ],
        out_specs=[
            pl.BlockSpec((bs, 64), lambda i: (i, 0)),
            pl.BlockSpec((bs, 64), lambda i: (i, 0)),
            pl.BlockSpec((bs, 64), lambda i: (i, 0)),
            pl.BlockSpec((4, bs, 16), lambda i: (0, i, 0)),
            pl.BlockSpec((4, bs, 16), lambda i: (0, i, 0)),
            pl.BlockSpec((4, bs, 16), lambda i: (0, i, 0)),
        ],
        out_shape=[cshape, cshape, cshape, mshape, mshape, mshape],
    )(edge_attr, w1s, b1s, WCs, cCs, WMs, cMs)


def _nodepre_body(h, WA, WB, WV, A, B, V):
    hv = h[...]
    dot = lambda x, y: jnp.dot(x, y, preferred_element_type=F32)
    A[...] = dot(hv, WA[...])
    B[...] = dot(hv, WB[...])
    Vf = dot(hv, WV[...])
    for qq in range(4):
        V[qq] = Vf[:, qq * 16:(qq + 1) * 16]


def _run_nodepre(h, WA, WB, WV):
    nb = 50
    bs = _N // nb
    return pl.pallas_call(
        _nodepre_body,
        grid=(nb,),
        in_specs=[
            pl.BlockSpec((bs, 64), lambda i: (i, 0)),
            pl.BlockSpec((64, 64), lambda i: (0, 0)),
            pl.BlockSpec((64, 64), lambda i: (0, 0)),
            pl.BlockSpec((64, 64), lambda i: (0, 0)),
        ],
        out_specs=[
            pl.BlockSpec((bs, 64), lambda i: (i, 0)),
            pl.BlockSpec((bs, 64), lambda i: (i, 0)),
            pl.BlockSpec((4, bs, 16), lambda i: (0, i, 0)),
        ],
        out_shape=[
            jax.ShapeDtypeStruct((_N, 64), F32),
            jax.ShapeDtypeStruct((_N, 64), F32),
            jax.ShapeDtypeStruct((4, _N, 16), F32),
        ],
    )(h, WA, WB, WV)


def _sc_scores_body(col_hbm, row_hbm, A_hbm, B_hbm, C_hbm, cons_hbm,
                    s_hbm, mparts_hbm,
                    m_tbl, cidx, ridx, abuf, bbuf, cbuf, sbuf, consts,
                    sem, sem2, sem3):
    core = lax.axis_index("c")
    sid = lax.axis_index("s")
    wid = sid * 2 + core

    pltpu.sync_copy(cons_hbm, consts)

    def init_body(i, _):
        m_tbl[pl.ds(i * 16, 16)] = jnp.full((16,), -3e38, F32)
        return 0
    lax.fori_loop(0, _N // 16, init_body, 0)

    w2v = tuple(consts[j, pl.ds(0, 16)] for j in range(4))
    lane = lax.iota(jnp.int32, 16)

    nt = 195 + jnp.where(wid < (_NCHUNK - 195 * _NTILE), 1, 0)

    def chunk_body(t, _):
        base = (wid + t * _NTILE) * _K
        pltpu.sync_copy(col_hbm.at[pl.ds(base, _K)], cidx)
        pltpu.sync_copy(row_hbm.at[pl.ds(base, _K)], ridx)
        ca = pltpu.async_copy(A_hbm.at[cidx], abuf, sem)
        cb = pltpu.async_copy(B_hbm.at[ridx], bbuf, sem2)
        cc = pltpu.async_copy(C_hbm.at[pl.ds(base, _K)], cbuf, sem3)
        ca.wait()
        cb.wait()
        cc.wait()

        def group_body(g, _):
            idxe = lane + g * 16
            s16 = consts[4, pl.ds(0, 16)]
            for f in range(64):
                f16 = jnp.full((16,), f, jnp.int32)
                u = (plsc.load_gather(abuf, [idxe, f16])
                     + plsc.load_gather(bbuf, [idxe, f16])
                     + plsc.load_gather(cbuf, [idxe, f16]))
                s16 = s16 + jnp.maximum(u, 0.0) * w2v[f // 16][f % 16]
            sbuf[pl.ds(g * 16, 16)] = s16
            c16 = cidx[pl.ds(g * 16, 16)]
            old = plsc.load_gather(m_tbl, [c16])
            plsc.store_scatter(m_tbl, [c16], jnp.maximum(old, s16))
            return 0
        lax.fori_loop(0, 8, group_body, 0)

        pltpu.sync_copy(sbuf, s_hbm.at[pl.ds(base, _K)])
        return 0
    lax.fori_loop(0, nt, chunk_body, 0)

    pltpu.sync_copy(m_tbl, mparts_hbm.at[wid])


def _run_sc_scores(col, row, A, B, C, cons):
    mesh = plsc.VectorSubcoreMesh(core_axis_name="c", subcore_axis_name="s")
    kern = functools.partial(
        pl.kernel,
        mesh=mesh,
        compiler_params=pltpu.CompilerParams(needs_layout_passes=False, use_tc_tiling_on_sc=False),
        out_type=[
            jax.ShapeDtypeStruct((_E,), F32),
            jax.ShapeDtypeStruct((_NTILE, _N), F32),
        ],
        scratch_types=[
            pltpu.VMEM((_N,), F32),
            pltpu.VMEM((_K,), jnp.int32),
            pltpu.VMEM((_K,), jnp.int32),
            pltpu.VMEM((_K, 64), F32),
            pltpu.VMEM((_K, 64), F32),
            pltpu.VMEM((_K, 64), F32),
            pltpu.VMEM((_K,), F32),
            pltpu.VMEM((8, 16), F32),
            pltpu.SemaphoreType.DMA,
            pltpu.SemaphoreType.DMA,
            pltpu.SemaphoreType.DMA,
        ],
    )(_sc_scores_body)
    return kern(col, row, A, B, C, cons)


_MSZ_A = 1568
_MSZ_B = 1392


def _sc_mmerge_body(mparts_hbm, m_hbm, mbuf, sem):
    core = lax.axis_index("c")
    sid = lax.axis_index("s")
    wid = sid * 2 + core

    def merge_range(start, size):
        pltpu.sync_copy(mparts_hbm.at[pl.ds(0, _NTILE), pl.ds(start, size)],
                        mbuf.at[pl.ds(0, _NTILE), pl.ds(0, size)])

        def red_body(i, _):
            acc = mbuf[0, pl.ds(i * 16, 16)]
            for k in range(1, _NTILE):
                acc = jnp.maximum(acc, mbuf[k, pl.ds(i * 16, 16)])
            mbuf[0, pl.ds(i * 16, 16)] = acc
            return 0
        lax.fori_loop(0, size // 16, red_body, 0)
        pltpu.sync_copy(mbuf.at[0, pl.ds(0, size)],
                        m_hbm.at[pl.ds(start, size)])

    @pl.when(wid < _NTILE - 1)
    def _():
        merge_range(wid * _MSZ_A, _MSZ_A)

    @pl.when(wid == _NTILE - 1)
    def _():
        merge_range((_NTILE - 1) * _MSZ_A, _MSZ_B)


def _run_sc_mmerge(mparts):
    mesh = plsc.VectorSubcoreMesh(core_axis_name="c", subcore_axis_name="s")
    kern = functools.partial(
        pl.kernel,
        mesh=mesh,
        compiler_params=pltpu.CompilerParams(needs_layout_passes=False, use_tc_tiling_on_sc=False),
        out_type=[jax.ShapeDtypeStruct((_N,), F32)],
        scratch_types=[
            pltpu.VMEM((_NTILE, _MSZ_A), F32),
            pltpu.SemaphoreType.DMA,
        ],
    )(_sc_mmerge_body)
    return kern(mparts)


def _sc_agg_body(col_hbm, row_hbm, s_all_hbm, m_hbm, V_hbm, M_hbm,
                 zeros16_hbm, zeros8_hbm,
                 T_hbm, S8_hbm,
                 m_tbl, cidx, ridx, sbuf, vbuf, mbuf, e8,
                 Tsh, Ssh, sem, sem2):
    core = lax.axis_index("c")
    sid = lax.axis_index("s")
    lane = lax.iota(jnp.int32, 16)

    pltpu.sync_copy(m_hbm, m_tbl)

    def z8_body(i, _):
        idx0 = lax.iota(jnp.int32, 16) + i * 16
        for c in range(8):
            plsc.store_scatter(e8, [idx0, jnp.full((16,), c, jnp.int32)],
                               jnp.zeros((16,), F32))
        return 0
    lax.fori_loop(0, 8, z8_body, 0)

    nt = 390 + jnp.where(sid < (_NCHUNK - 390 * 16), 1, 0)

    for ph in range(2):
        q = core * 2 + ph
        @pl.when(sid < 15)
        def _():
            pltpu.sync_copy(zeros16_hbm, Tsh.at[pl.ds(sid * _ROWS_A, _ROWS_A)])

        @pl.when(sid == 15)
        def _():
            pltpu.sync_copy(zeros16_hbm.at[pl.ds(0, _ROWS_B)],
                            Tsh.at[pl.ds(15 * _ROWS_A, _ROWS_B)])
        if ph == 0:
            @pl.when(sid < 15)
            def _():
                pltpu.sync_copy(zeros8_hbm,
                                Ssh.at[pl.ds(sid * _ROWS_A, _ROWS_A)])

            @pl.when(sid == 15)
            def _():
                pltpu.sync_copy(zeros8_hbm.at[pl.ds(0, _ROWS_B)],
                                Ssh.at[pl.ds(15 * _ROWS_A, _ROWS_B)])
        plsc.subcore_barrier()

        def chunk_body(t, _):
            base = (sid + t * 16) * _K
            pltpu.sync_copy(col_hbm.at[pl.ds(base, _K)], cidx)
            pltpu.sync_copy(row_hbm.at[pl.ds(base, _K)], ridx)
            pltpu.sync_copy(s_all_hbm.at[pl.ds(base, _K)], sbuf)
            cv = pltpu.async_copy(V_hbm.at[q].at[ridx], vbuf, sem)
            cm = pltpu.async_copy(M_hbm.at[q, pl.ds(base, _K)], mbuf, sem2)
            cv.wait()
            cm.wait()

            def group_body(g, _):
                c16 = cidx[pl.ds(g * 16, 16)]
                s16 = sbuf[pl.ds(g * 16, 16)]
                mg = plsc.load_gather(m_tbl, [c16])
                e16 = jnp.exp(s16 - mg)
                if ph == 0:
                    plsc.store_scatter(e8, [lane + g * 16,
                                            jnp.zeros((16,), jnp.int32)], e16)
                for k in range(16):
                    i = g * 16 + k
                    ev = jnp.full((16,), e16[k], F32)
                    u = vbuf[i, pl.ds(0, 16)] + mbuf[i, pl.ds(0, 16)]
                    vbuf[i, pl.ds(0, 16)] = ev * (u / (1.0 + jnp.exp(-u)))
                return 0
            lax.fori_loop(0, 8, group_body, 0)

            pltpu.sync_copy(vbuf, Tsh.at[cidx], add=True)
            if ph == 0:
                pltpu.sync_copy(e8, Ssh.at[cidx], add=True)
            return 0
        lax.fori_loop(0, nt, chunk_body, 0)

        plsc.subcore_barrier()

        @pl.when(sid < 15)
        def _():
            r0 = sid * _ROWS_A
            pltpu.sync_copy(Tsh.at[pl.ds(r0, _ROWS_A)],
                            T_hbm.at[q, pl.ds(r0, _ROWS_A)])

        @pl.when(sid == 15)
        def _():
            r0 = 15 * _ROWS_A
            pltpu.sync_copy(Tsh.at[pl.ds(r0, _ROWS_B)],
                            T_hbm.at[q, pl.ds(r0, _ROWS_B)])

        if ph == 0:
            @pl.when((sid < 15) & (core == 1))
            def _():
                r0 = sid * _ROWS_A
                pltpu.sync_copy(Ssh.at[pl.ds(r0, _ROWS_A)],
                                S8_hbm.at[pl.ds(r0, _ROWS_A)])

            @pl.when((sid == 15) & (core == 1))
            def _():
                pltpu.sync_copy(Ssh.at[pl.ds(15 * _ROWS_A, _ROWS_B)],
                                S8_hbm.at[pl.ds(15 * _ROWS_A, _ROWS_B)])
        plsc.subcore_barrier()


def _run_sc_agg(col, row, s_all, m, V, M, zeros16, zeros8):
    mesh = plsc.VectorSubcoreMesh(core_axis_name="c", subcore_axis_name="s")
    kern = functools.partial(
        pl.kernel,
        mesh=mesh,
        compiler_params=pltpu.CompilerParams(needs_layout_passes=False, use_tc_tiling_on_sc=False),
        out_type=[
            jax.ShapeDtypeStruct((4, _N, 16), F32),
            jax.ShapeDtypeStruct((_N, 8), F32),
        ],
        scratch_types=[
            pltpu.VMEM((_N,), F32),
            pltpu.VMEM((_K,), jnp.int32),
            pltpu.VMEM((_K,), jnp.int32),
            pltpu.VMEM((_K,), F32),
            pltpu.VMEM((_K, 16), F32),
            pltpu.VMEM((_K, 16), F32),
            pltpu.VMEM((_K, 8), F32),
            pltpu.VMEM_SHARED((_N, 16), F32),
            pltpu.VMEM_SHARED((_N, 8), F32),
            pltpu.SemaphoreType.DMA,
            pltpu.SemaphoreType.DMA,
        ],
    )(_sc_agg_body)
    return kern(col, row, s_all, m, V, M, zeros16, zeros8)


def _post_body(T, S8, h, Wm2, bm2, hnew):
    S = S8[...][:, 0:1]
    Tc = jnp.concatenate([T[0], T[1], T[2], T[3]], axis=-1)
    w = Tc / (S + 1e-16)
    agg = (jnp.dot(w, Wm2[...], preferred_element_type=F32)
           + jnp.where(S > 0.0, 1.0, 0.0) * bm2[...])
    hnew[...] = h[...] + agg


def _run_post(T, S8, h, Wm2, bm2):
    nb = 50
    bs = _N // nb
    return pl.pallas_call(
        _post_body,
        grid=(nb,),
        in_specs=[
            pl.BlockSpec((4, bs, 16), lambda i: (0, i, 0)),
            pl.BlockSpec((bs, 8), lambda i: (i, 0)),
            pl.BlockSpec((bs, 64), lambda i: (i, 0)),
            pl.BlockSpec((64, 64), lambda i: (0, 0)),
            pl.BlockSpec((1, 64), lambda i: (0, 0)),
        ],
        out_specs=pl.BlockSpec((bs, 64), lambda i: (i, 0)),
        out_shape=jax.ShapeDtypeStruct((_N, 64), F32),
    )(T, S8, h, Wm2, bm2.reshape(1, -1))


def _pool_body(h, bcol, acc_out, acc):
    i = pl.program_id(0)

    @pl.when(i == 0)
    def _():
        acc[...] = jnp.zeros_like(acc)

    oh = (bcol[...] == lax.broadcasted_iota(jnp.int32, (h.shape[0], _NG), 1))
    ohf = oh.astype(F32)
    hh = jnp.concatenate([h[...], jnp.ones_like(h[...])], axis=-1)
    acc[...] += lax.dot_general(ohf, hh, (((0,), (0,)), ((), ())),
                                preferred_element_type=F32)

    @pl.when(i == pl.num_programs(0) - 1)
    def _():
        acc_out[...] = acc[...]


def _run_pool(h, batch):
    nb = 50
    bs = _N // nb
    return pl.pallas_call(
        _pool_body,
        grid=(nb,),
        in_specs=[
            pl.BlockSpec((bs, 64), lambda i: (i, 0)),
            pl.BlockSpec((bs, 1), lambda i: (i, 0)),
        ],
        out_specs=pl.BlockSpec((_NG, 128), lambda i: (0, 0)),
        out_shape=jax.ShapeDtypeStruct((_NG, 128), F32),
        scratch_shapes=[pltpu.VMEM((_NG, 128), F32)],
    )(h, batch.reshape(-1, 1))


def _heads_body(acc, *refs):
    hsum = acc[...][:, 0:64]
    cnt = acc[...][:, 64:65]
    hg = hsum / jnp.maximum(cnt, 1.0)
    nw = [3, 4, 3, 3, 3]
    order = [0, 1, 1, 0, 0]
    pos = 0
    preds = []
    ins = refs[:-1]
    out = refs[-1]
    for hi in range(5):
        z = hg
        depth = 4 if order[hi] else 3
        for d in range(depth):
            W = ins[pos][...]
            b = ins[pos + 1][...]
            pos += 2
            z = jnp.dot(z, W, preferred_element_type=F32) + b
            if d < depth - 1:
                z = _silu(z)
        preds.append(z)
    out[...] = jnp.concatenate(preds, axis=-1)


def _run_heads(acc, heads):
    order = ['neff', 'Aeff', 'NL', 'Disp', 'GVD']
    args = [acc]
    for name in order:
        for (W, b) in heads[name]:
            args.append(W)
            args.append(b.reshape(1, -1))
    return pl.pallas_call(
        _heads_body,
        out_shape=jax.ShapeDtypeStruct((_NG, 5), F32),
    )(*args)


def kernel(x, pos, graph_features, batch, edge_index, edge_attr, params):
    del pos
    (WAs, WBs, WVs, WCs, cCs, WMs, cMs, cons, G) = _run_prep(
        params, graph_features)
    Wne, bne = params['node_enc']
    h = _run_enc(x, batch, Wne, bne, G)

    blocks = params['blocks']
    w1s = jnp.stack([b['edge1'][0] for b in blocks])
    b1s = jnp.stack([b['edge1'][1].reshape(1, -1) for b in blocks])
    C0, C1, C2, M0, M1, M2 = _run_edgepre(edge_attr, w1s, b1s,
                                          WCs, cCs, WMs, cMs)
    Cs = (C0, C1, C2)
    Ms = (M0, M1, M2)

    col = edge_index[1]
    row = edge_index[0]
    zeros16 = jnp.zeros((_ROWS_A, 16), F32)
    zeros8 = jnp.zeros((_ROWS_A, 8), F32)

    for l in range(_NL):
        A, B, V = _run_nodepre(h, WAs[l], WBs[l], WVs[l])
        s_all, mparts = _run_sc_scores(col, row, A, B, Cs[l], cons[l])
        m, = _run_sc_mmerge(mparts)
        T, S8 = _run_sc_agg(col, row, s_all, m, V, Ms[l],
                            zeros16, zeros8)
        Wm2, bm2 = blocks[l]['msg2']
        h = _run_post(T, S8, h, Wm2, bm2)

    acc = _run_pool(h, batch)
    return _run_heads(acc, params['heads'])


# speedup vs baseline: 2.5400x; 1.1884x over previous
import functools

import jax
import jax.numpy as jnp
from jax import lax
from jax.experimental import pallas as pl
from jax.experimental.pallas import tpu as pltpu
from jax.experimental.pallas import tpu_sc as plsc

F32 = jnp.float32

_N = 50000
_E = 800000
_HID = 64
_NG = 8
_NL = 3

_NTILE = 32
_K1 = 256
_K2 = 640
_ROWS_A = 3200
_ROWS_B = 2000


def _silu(u):
    return u / (1.0 + jnp.exp(-u))


def _prep_body(Wqs, bqs, Wks, bks, Wvs, bvs, Wa1s, ba1s, wa2s, ba2s,
               We2s, be2s, Wm1s, bm1s, gf, Wge, bge,
               WAs, WBs, WVs, WCs, cCs, WMs, cMs, cons, G):
    for l in range(_NL):
        Wq, bq = Wqs[l], bqs[l]
        Wk, bk = Wks[l], bks[l]
        Wv, bv = Wvs[l], bvs[l]
        Wa1 = Wa1s[l]
        Wa1_q, Wa1_k, Wa1_e = Wa1[0:64, :], Wa1[64:128, :], Wa1[128:192, :]
        Wm1 = Wm1s[l]
        Wm1_v, Wm1_e = Wm1[0:64, :], Wm1[64:128, :]
        We2, be2 = We2s[l], be2s[l]
        dot = lambda x, y: jnp.dot(x, y, preferred_element_type=F32)
        WAs[l] = dot(Wq, Wa1_q)
        WBs[l] = dot(Wk, Wa1_k)
        WVs[l] = dot(Wv, Wm1_v)
        WCs[l] = dot(We2, Wa1_e)
        cCs[l] = ba1s[l] + dot(bq, Wa1_q) + dot(bk, Wa1_k) + dot(be2, Wa1_e)
        WMs[l] = dot(We2, Wm1_e)
        cMs[l] = bm1s[l] + dot(bv, Wm1_v) + dot(be2, Wm1_e)
        w2 = wa2s[l].reshape(4, 16)
        b2 = jnp.full((4, 16), ba2s[l][0, 0], F32)
        cons[l] = jnp.concatenate([w2, b2], axis=0)
    G[...] = jnp.dot(gf[...], Wge[...], preferred_element_type=F32) + bge[...]


def _run_prep(params, graph_features):
    blocks = params['blocks']
    st = lambda key: jnp.stack([b[key][0] for b in blocks])
    stb = lambda key: jnp.stack([b[key][1].reshape(1, -1) for b in blocks])
    Wge, bge = params['graph_enc']
    outs = pl.pallas_call(
        _prep_body,
        out_shape=[
            jax.ShapeDtypeStruct((_NL, 64, 64), F32),
            jax.ShapeDtypeStruct((_NL, 64, 64), F32),
            jax.ShapeDtypeStruct((_NL, 64, 64), F32),
            jax.ShapeDtypeStruct((_NL, 64, 64), F32),
            jax.ShapeDtypeStruct((_NL, 1, 64), F32),
            jax.ShapeDtypeStruct((_NL, 64, 64), F32),
            jax.ShapeDtypeStruct((_NL, 1, 64), F32),
            jax.ShapeDtypeStruct((_NL, 8, 16), F32),
            jax.ShapeDtypeStruct((_NG, 64), F32),
        ],
    )(st('q'), stb('q'), st('k'), stb('k'), st('v'), stb('v'),
      st('att1'), stb('att1'), st('att2'), stb('att2'),
      st('edge2'), stb('edge2'), st('msg1'), stb('msg1'),
      graph_features, Wge, bge.reshape(1, -1))
    return outs


def _enc_body(x, bcol, Wne, bne, G, h):
    oh = (bcol[...] == lax.broadcasted_iota(jnp.int32, (x.shape[0], _NG), 1))
    h[...] = (jnp.dot(x[...], Wne[...], preferred_element_type=F32) + bne[...]
              + jnp.dot(oh.astype(F32), G[...], preferred_element_type=F32))


def _run_enc(x, batch, Wne, bne, G):
    nb = 50
    bs = _N // nb
    return pl.pallas_call(
        _enc_body,
        grid=(nb,),
        in_specs=[
            pl.BlockSpec((bs, 128), lambda i: (i, 0)),
            pl.BlockSpec((bs, 1), lambda i: (i, 0)),
            pl.BlockSpec((128, 64), lambda i: (0, 0)),
            pl.BlockSpec((1, 64), lambda i: (0, 0)),
            pl.BlockSpec((_NG, 64), lambda i: (0, 0)),
        ],
        out_specs=pl.BlockSpec((bs, 64), lambda i: (i, 0)),
        out_shape=jax.ShapeDtypeStruct((_N, 64), F32),
    )(x, batch.reshape(-1, 1), Wne, bne.reshape(1, -1), G)


def _edgepre_body(a, w1s, b1s, WCs, cCs, WMs, cMs, C0, C1, C2, M0, M1, M2):
    Couts = (C0, C1, C2)
    Mouts = (M0, M1, M2)
    for l in range(_NL):
        r = jnp.maximum(a[...] * w1s[l] + b1s[l], 0.0)
        Couts[l][...] = (jnp.dot(r, WCs[l], preferred_element_type=F32)
                         + cCs[l])
        M = jnp.dot(r, WMs[l], preferred_element_type=F32) + cMs[l]
        for qq in range(4):
            Mouts[l][qq] = M[:, qq * 16:(qq + 1) * 16]


def _run_edgepre(edge_attr, w1s, b1s, WCs, cCs, WMs, cMs):
    nb = 400
    bs = _E // nb
    cshape = jax.ShapeDtypeStruct((_E, 64), F32)
    mshape = jax.ShapeDtypeStruct((4, _E, 16), F32)
    full = lambda s: pl.BlockSpec(s, lambda i: tuple(0 for _ in s))
    return pl.pallas_call(
        _edgepre_body,
        grid=(nb,),
        in_specs=[
            pl.BlockSpec((bs, 1), lambda i: (i, 0)),
            full((_NL, 1, 64)), full((_NL, 1, 64)),
            full((_NL, 64, 64)), full((_NL, 1, 64)),
            full((_NL, 64, 64)), full((_NL, 1, 64)),
        ],
        out_specs=[
            pl.BlockSpec((bs, 64), lambda i: (i, 0)),
            pl.BlockSpec((bs, 64), lambda i: (i, 0)),
            pl.BlockSpec((bs, 64), lambda i: (i, 0)),
            pl.BlockSpec((4, bs, 16), lambda i: (0, i, 0)),
            pl.BlockSpec((4, bs, 16), lambda i: (0, i, 0)),
            pl.BlockSpec((4, bs, 16), lambda i: (0, i, 0)),
        ],
        out_shape=[cshape, cshape, cshape, mshape, mshape, mshape],
    )(edge_attr, w1s, b1s, WCs, cCs, WMs, cMs)


def _nodepre_body(h, WA, WB, WV, A, B, V):
    hv = h[...]
    dot = lambda x, y: jnp.dot(x, y, preferred_element_type=F32)
    A[...] = dot(hv, WA[...])
    B[...] = dot(hv, WB[...])
    Vf = dot(hv, WV[...])
    for qq in range(4):
        V[qq] = Vf[:, qq * 16:(qq + 1) * 16]


def _run_nodepre(h, WA, WB, WV):
    nb = 50
    bs = _N // nb
    return pl.pallas_call(
        _nodepre_body,
        grid=(nb,),
        in_specs=[
            pl.BlockSpec((bs, 64), lambda i: (i, 0)),
            pl.BlockSpec((64, 64), lambda i: (0, 0)),
            pl.BlockSpec((64, 64), lambda i: (0, 0)),
            pl.BlockSpec((64, 64), lambda i: (0, 0)),
        ],
        out_specs=[
            pl.BlockSpec((bs, 64), lambda i: (i, 0)),
            pl.BlockSpec((bs, 64), lambda i: (i, 0)),
            pl.BlockSpec((4, bs, 16), lambda i: (0, i, 0)),
        ],
        out_shape=[
            jax.ShapeDtypeStruct((_N, 64), F32),
            jax.ShapeDtypeStruct((_N, 64), F32),
            jax.ShapeDtypeStruct((4, _N, 16), F32),
        ],
    )(h, WA, WB, WV)


def _sc_scores_body(col_hbm, row_hbm, A_hbm, B_hbm, C_hbm, cons_hbm,
                    s_hbm, mparts_hbm,
                    m_tbl, cidx, ridx, abuf, bbuf, cbuf, sbuf, consts,
                    isem, gsem):
    core = lax.axis_index("c")
    sid = lax.axis_index("s")
    wid = sid * 2 + core

    pltpu.sync_copy(cons_hbm, consts)

    def init_body(i, _):
        m_tbl[pl.ds(i * 16, 16)] = jnp.full((16,), -3e38, F32)
        return 0
    lax.fori_loop(0, _N // 16, init_body, 0)

    w2v = tuple(consts[j, pl.ds(0, 16)] for j in range(4))
    lane = lax.iota(jnp.int32, 16)

    nc = _E // _K1
    ntbase = nc // _NTILE
    nt = ntbase + jnp.where(wid < (nc - ntbase * _NTILE), 1, 0)

    def chunk_body(t, _):
        base = (wid + t * _NTILE) * _K1
        for j in range(2):
            pltpu.async_copy(col_hbm.at[pl.ds(base + j * 128, 128)],
                             cidx.at[j], isem)
            pltpu.async_copy(row_hbm.at[pl.ds(base + j * 128, 128)],
                             ridx.at[j], isem)
        for _i in range(4):
            pltpu.make_async_copy(col_hbm.at[pl.ds(base, 128)],
                                  cidx.at[0], isem).wait()
        for j in range(2):
            pltpu.async_copy(A_hbm.at[cidx.at[j]],
                             abuf.at[pl.ds(j * 128, 128)], gsem)
            pltpu.async_copy(B_hbm.at[ridx.at[j]],
                             bbuf.at[pl.ds(j * 128, 128)], gsem)
        pltpu.async_copy(C_hbm.at[pl.ds(base, _K1)], cbuf, gsem)
        for j in range(2):
            pltpu.make_async_copy(A_hbm.at[cidx.at[j]],
                                  abuf.at[pl.ds(j * 128, 128)], gsem).wait()
            pltpu.make_async_copy(B_hbm.at[ridx.at[j]],
                                  bbuf.at[pl.ds(j * 128, 128)], gsem).wait()
        pltpu.make_async_copy(C_hbm.at[pl.ds(base, _K1)], cbuf, gsem).wait()

        def group_body(g, _):
            idxe = lane + g * 16
            s16 = consts[4, pl.ds(0, 16)]
            for f in range(64):
                f16 = jnp.full((16,), f, jnp.int32)
                u = (plsc.load_gather(abuf, [idxe, f16])
                     + plsc.load_gather(bbuf, [idxe, f16])
                     + plsc.load_gather(cbuf, [idxe, f16]))
                s16 = s16 + jnp.maximum(u, 0.0) * w2v[f // 16][f % 16]
            sbuf[pl.ds(g * 16, 16)] = s16
            c16 = cidx[g // 8, pl.ds((g % 8) * 16, 16)]
            old = plsc.load_gather(m_tbl, [c16])
            plsc.store_scatter(m_tbl, [c16], jnp.maximum(old, s16))
            return 0
        lax.fori_loop(0, _K1 // 16, group_body, 0)

        pltpu.sync_copy(sbuf, s_hbm.at[pl.ds(base, _K1)])
        return 0
    lax.fori_loop(0, nt, chunk_body, 0)

    pltpu.sync_copy(m_tbl, mparts_hbm.at[wid])


def _run_sc_scores(col, row, A, B, C, cons):
    mesh = plsc.VectorSubcoreMesh(core_axis_name="c", subcore_axis_name="s")
    kern = functools.partial(
        pl.kernel,
        mesh=mesh,
        compiler_params=pltpu.CompilerParams(needs_layout_passes=False, use_tc_tiling_on_sc=False),
        out_type=[
            jax.ShapeDtypeStruct((_E,), F32),
            jax.ShapeDtypeStruct((_NTILE, _N), F32),
        ],
        scratch_types=[
            pltpu.VMEM((_N,), F32),
            pltpu.VMEM((2, 128), jnp.int32),
            pltpu.VMEM((2, 128), jnp.int32),
            pltpu.VMEM((_K1, 64), F32),
            pltpu.VMEM((_K1, 64), F32),
            pltpu.VMEM((_K1, 64), F32),
            pltpu.VMEM((_K1,), F32),
            pltpu.VMEM((8, 16), F32),
            pltpu.SemaphoreType.DMA,
            pltpu.SemaphoreType.DMA,
        ],
    )(_sc_scores_body)
    return kern(col, row, A, B, C, cons)


_MSZ_A = 1568
_MSZ_B = 1392


def _sc_mmerge_body(mparts_hbm, m_hbm, mbuf, sem):
    core = lax.axis_index("c")
    sid = lax.axis_index("s")
    wid = sid * 2 + core

    def merge_range(start, size):
        pltpu.sync_copy(mparts_hbm.at[pl.ds(0, _NTILE), pl.ds(start, size)],
                        mbuf.at[pl.ds(0, _NTILE), pl.ds(0, size)])

        def red_body(i, _):
            acc = mbuf[0, pl.ds(i * 16, 16)]
            for k in range(1, _NTILE):
                acc = jnp.maximum(acc, mbuf[k, pl.ds(i * 16, 16)])
            mbuf[0, pl.ds(i * 16, 16)] = acc
            return 0
        lax.fori_loop(0, size // 16, red_body, 0)
        pltpu.sync_copy(mbuf.at[0, pl.ds(0, size)],
                        m_hbm.at[pl.ds(start, size)])

    @pl.when(wid < _NTILE - 1)
    def _():
        merge_range(wid * _MSZ_A, _MSZ_A)

    @pl.when(wid == _NTILE - 1)
    def _():
        merge_range((_NTILE - 1) * _MSZ_A, _MSZ_B)


def _run_sc_mmerge(mparts):
    mesh = plsc.VectorSubcoreMesh(core_axis_name="c", subcore_axis_name="s")
    kern = functools.partial(
        pl.kernel,
        mesh=mesh,
        compiler_params=pltpu.CompilerParams(needs_layout_passes=False, use_tc_tiling_on_sc=False),
        out_type=[jax.ShapeDtypeStruct((_N,), F32)],
        scratch_types=[
            pltpu.VMEM((_NTILE, _MSZ_A), F32),
            pltpu.SemaphoreType.DMA,
        ],
    )(_sc_mmerge_body)
    return kern(mparts)


def _sc_agg_body(col_hbm, row_hbm, s_all_hbm, m_hbm, V_hbm, M_hbm,
                 zeros16_hbm,
                 T_hbm, S_hbm,
                 m_tbl, cidx, ridx, sbuf, vbuf, mbuf, Tsh, isem, gsem):
    core = lax.axis_index("c")
    sid = lax.axis_index("s")
    lane = lax.iota(jnp.int32, 16)

    pltpu.sync_copy(m_hbm, m_tbl)

    nc = _E // _K2
    ntb = nc // 16
    nt = ntb + jnp.where(sid < (nc - ntb * 16), 1, 0)

    def zero_tsh():
        @pl.when(sid < 15)
        def _():
            pltpu.sync_copy(zeros16_hbm, Tsh.at[pl.ds(sid * _ROWS_A, _ROWS_A)])

        @pl.when(sid == 15)
        def _():
            pltpu.sync_copy(zeros16_hbm.at[pl.ds(0, _ROWS_B)],
                            Tsh.at[pl.ds(15 * _ROWS_A, _ROWS_B)])

    def copy_tsh_out(dst):
        @pl.when(sid < 15)
        def _():
            r0 = sid * _ROWS_A
            pltpu.sync_copy(Tsh.at[pl.ds(r0, _ROWS_A)],
                            dst.at[pl.ds(r0, _ROWS_A)])

        @pl.when(sid == 15)
        def _():
            r0 = 15 * _ROWS_A
            pltpu.sync_copy(Tsh.at[pl.ds(r0, _ROWS_B)],
                            dst.at[pl.ds(r0, _ROWS_B)])

    def load_idx(base, want_row):
        for j in range(5):
            pltpu.async_copy(col_hbm.at[pl.ds(base + j * 128, 128)],
                             cidx.at[j], isem)
            if want_row:
                pltpu.async_copy(row_hbm.at[pl.ds(base + j * 128, 128)],
                                 ridx.at[j], isem)
        pltpu.async_copy(s_all_hbm.at[pl.ds(base, _K2)], sbuf, isem)
        n = 10 if want_row else 5
        for _i in range(n):
            pltpu.make_async_copy(col_hbm.at[pl.ds(base, 128)],
                                  cidx.at[0], isem).wait()
        pltpu.make_async_copy(s_all_hbm.at[pl.ds(base, _K2)], sbuf,
                              isem).wait()

    def e_of_group(g):
        c16 = cidx[g // 8, pl.ds((g % 8) * 16, 16)]
        s16 = sbuf[pl.ds(g * 16, 16)]
        mg = plsc.load_gather(m_tbl, [c16])
        return jnp.exp(s16 - mg)

    for ph in range(2):
        q = core * 2 + ph
        zero_tsh()
        plsc.subcore_barrier()

        def chunk_body(t, _):
            base = (sid + t * 16) * _K2
            load_idx(base, True)
            for j in range(5):
                pltpu.async_copy(V_hbm.at[q].at[ridx.at[j]],
                                 vbuf.at[pl.ds(j * 128, 128)], gsem)
            pltpu.async_copy(M_hbm.at[q, pl.ds(base, _K2)], mbuf, gsem)
            for j in range(5):
                pltpu.make_async_copy(V_hbm.at[q].at[ridx.at[j]],
                                      vbuf.at[pl.ds(j * 128, 128)],
                                      gsem).wait()
            pltpu.make_async_copy(M_hbm.at[q, pl.ds(base, _K2)], mbuf,
                                  gsem).wait()

            def group_body(g, _):
                e16 = e_of_group(g)
                for k in range(16):
                    i = g * 16 + k
                    ev = jnp.full((16,), e16[k], F32)
                    u = vbuf[i, pl.ds(0, 16)] + mbuf[i, pl.ds(0, 16)]
                    vbuf[i, pl.ds(0, 16)] = ev * (u / (1.0 + jnp.exp(-u)))
                return 0
            lax.fori_loop(0, _K2 // 16, group_body, 0)

            for j in range(5):
                pltpu.sync_copy(vbuf.at[pl.ds(j * 128, 128)],
                                Tsh.at[cidx.at[j]], add=True)
            return 0
        lax.fori_loop(0, nt, chunk_body, 0)

        plsc.subcore_barrier()
        copy_tsh_out(T_hbm.at[q])
        plsc.subcore_barrier()

    def zv_body(i, _):
        vbuf[i, pl.ds(0, 16)] = jnp.zeros((16,), F32)
        return 0
    lax.fori_loop(0, _K2, zv_body, 0)
    zero_tsh()
    plsc.subcore_barrier()

    wid = core * 16 + sid
    ntb3 = nc // _NTILE
    nt3 = ntb3 + jnp.where(wid < (nc - ntb3 * _NTILE), 1, 0)

    def s_chunk(t, _):
        base = (wid + t * _NTILE) * _K2
        load_idx(base, False)

        def sg_body(g, _):
            e16 = e_of_group(g)
            plsc.store_scatter(vbuf, [lane + g * 16,
                                      jnp.zeros((16,), jnp.int32)], e16)
            return 0
        lax.fori_loop(0, _K2 // 16, sg_body, 0)

        for j in range(5):
            pltpu.sync_copy(vbuf.at[pl.ds(j * 128, 128)],
                            Tsh.at[cidx.at[j]], add=True)
        return 0
    lax.fori_loop(0, nt3, s_chunk, 0)

    plsc.subcore_barrier()
    copy_tsh_out(S_hbm.at[core])


def _run_sc_agg(col, row, s_all, m, V, M, zeros16):
    mesh = plsc.VectorSubcoreMesh(core_axis_name="c", subcore_axis_name="s")
    kern = functools.partial(
        pl.kernel,
        mesh=mesh,
        compiler_params=pltpu.CompilerParams(needs_layout_passes=False, use_tc_tiling_on_sc=False),
        out_type=[
            jax.ShapeDtypeStruct((4, _N, 16), F32),
            jax.ShapeDtypeStruct((2, _N, 16), F32),
        ],
        scratch_types=[
            pltpu.VMEM((_N,), F32),
            pltpu.VMEM((5, 128), jnp.int32),
            pltpu.VMEM((5, 128), jnp.int32),
            pltpu.VMEM((_K2,), F32),
            pltpu.VMEM((_K2, 16), F32),
            pltpu.VMEM((_K2, 16), F32),
            pltpu.VMEM_SHARED((_N, 16), F32),
            pltpu.SemaphoreType.DMA,
            pltpu.SemaphoreType.DMA,
        ],
    )(_sc_agg_body)
    return kern(col, row, s_all, m, V, M, zeros16)


def _post_body(T, Sq, h, Wm2, bm2, hnew):
    S = Sq[0][:, 0:1] + Sq[1][:, 0:1]
    Tc = jnp.concatenate([T[0], T[1], T[2], T[3]], axis=-1)
    w = Tc / (S + 1e-16)
    agg = (jnp.dot(w, Wm2[...], preferred_element_type=F32)
           + jnp.where(S > 0.0, 1.0, 0.0) * bm2[...])
    hnew[...] = h[...] + agg


def _run_post(T, Sq, h, Wm2, bm2):
    nb = 50
    bs = _N // nb
    return pl.pallas_call(
        _post_body,
        grid=(nb,),
        in_specs=[
            pl.BlockSpec((4, bs, 16), lambda i: (0, i, 0)),
            pl.BlockSpec((2, bs, 16), lambda i: (0, i, 0)),
            pl.BlockSpec((bs, 64), lambda i: (i, 0)),
            pl.BlockSpec((64, 64), lambda i: (0, 0)),
            pl.BlockSpec((1, 64), lambda i: (0, 0)),
        ],
        out_specs=pl.BlockSpec((bs, 64), lambda i: (i, 0)),
        out_shape=jax.ShapeDtypeStruct((_N, 64), F32),
    )(T, Sq, h, Wm2, bm2.reshape(1, -1))


def _pool_body(h, bcol, acc_out, acc):
    i = pl.program_id(0)

    @pl.when(i == 0)
    def _():
        acc[...] = jnp.zeros_like(acc)

    oh = (bcol[...] == lax.broadcasted_iota(jnp.int32, (h.shape[0], _NG), 1))
    ohf = oh.astype(F32)
    hh = jnp.concatenate([h[...], jnp.ones_like(h[...])], axis=-1)
    acc[...] += lax.dot_general(ohf, hh, (((0,), (0,)), ((), ())),
                                preferred_element_type=F32)

    @pl.when(i == pl.num_programs(0) - 1)
    def _():
        acc_out[...] = acc[...]


def _run_pool(h, batch):
    nb = 50
    bs = _N // nb
    return pl.pallas_call(
        _pool_body,
        grid=(nb,),
        in_specs=[
            pl.BlockSpec((bs, 64), lambda i: (i, 0)),
            pl.BlockSpec((bs, 1), lambda i: (i, 0)),
        ],
        out_specs=pl.BlockSpec((_NG, 128), lambda i: (0, 0)),
        out_shape=jax.ShapeDtypeStruct((_NG, 128), F32),
        scratch_shapes=[pltpu.VMEM((_NG, 128), F32)],
    )(h, batch.reshape(-1, 1))


def _heads_body(acc, *refs):
    hsum = acc[...][:, 0:64]
    cnt = acc[...][:, 64:65]
    hg = hsum / jnp.maximum(cnt, 1.0)
    nw = [3, 4, 3, 3, 3]
    order = [0, 1, 1, 0, 0]
    pos = 0
    preds = []
    ins = refs[:-1]
    out = refs[-1]
    for hi in range(5):
        z = hg
        depth = 4 if order[hi] else 3
        for d in range(depth):
            W = ins[pos][...]
            b = ins[pos + 1][...]
            pos += 2
            z = jnp.dot(z, W, preferred_element_type=F32) + b
            if d < depth - 1:
                z = _silu(z)
        preds.append(z)
    out[...] = jnp.concatenate(preds, axis=-1)


def _run_heads(acc, heads):
    order = ['neff', 'Aeff', 'NL', 'Disp', 'GVD']
    args = [acc]
    for name in order:
        for (W, b) in heads[name]:
            args.append(W)
            args.append(b.reshape(1, -1))
    return pl.pallas_call(
        _heads_body,
        out_shape=jax.ShapeDtypeStruct((_NG, 5), F32),
    )(*args)


def kernel(x, pos, graph_features, batch, edge_index, edge_attr, params):
    del pos
    (WAs, WBs, WVs, WCs, cCs, WMs, cMs, cons, G) = _run_prep(
        params, graph_features)
    Wne, bne = params['node_enc']
    h = _run_enc(x, batch, Wne, bne, G)

    blocks = params['blocks']
    w1s = jnp.stack([b['edge1'][0] for b in blocks])
    b1s = jnp.stack([b['edge1'][1].reshape(1, -1) for b in blocks])
    C0, C1, C2, M0, M1, M2 = _run_edgepre(edge_attr, w1s, b1s,
                                          WCs, cCs, WMs, cMs)
    Cs = (C0, C1, C2)
    Ms = (M0, M1, M2)

    col = edge_index[1]
    row = edge_index[0]
    zeros16 = jnp.zeros((_ROWS_A, 16), F32)

    for l in range(_NL):
        A, B, V = _run_nodepre(h, WAs[l], WBs[l], WVs[l])
        s_all, mparts = _run_sc_scores(col, row, A, B, Cs[l], cons[l])
        m, = _run_sc_mmerge(mparts)
        T, Sq = _run_sc_agg(col, row, s_all, m, V, Ms[l], zeros16)
        Wm2, bm2 = blocks[l]['msg2']
        h = _run_post(T, Sq, h, Wm2, bm2)

    acc = _run_pool(h, batch)
    return _run_heads(acc, params['heads'])


# speedup vs baseline: 3.4302x; 1.3505x over previous
import functools

import jax
import jax.numpy as jnp
from jax import lax
from jax.experimental import pallas as pl
from jax.experimental.pallas import tpu as pltpu
from jax.experimental.pallas import tpu_sc as plsc

F32 = jnp.float32

_N = 50000
_E = 800000
_HID = 64
_NG = 8
_NL = 3

_NTILE = 32
_K1 = 256
_K2 = 640
_ROWS_A = 3200
_ROWS_B = 2000


def _silu(u):
    return u / (1.0 + jnp.exp(-u))


def _prep_body(Wqs, bqs, Wks, bks, Wvs, bvs, Wa1s, ba1s, wa2s, ba2s,
               We2s, be2s, Wm1s, bm1s, gf, Wge, bge,
               WAs, WBs, WVs, WCs, cCs, WMs, cMs, cons, G):
    for l in range(_NL):
        Wq, bq = Wqs[l], bqs[l]
        Wk, bk = Wks[l], bks[l]
        Wv, bv = Wvs[l], bvs[l]
        Wa1 = Wa1s[l]
        Wa1_q, Wa1_k, Wa1_e = Wa1[0:64, :], Wa1[64:128, :], Wa1[128:192, :]
        Wm1 = Wm1s[l]
        Wm1_v, Wm1_e = Wm1[0:64, :], Wm1[64:128, :]
        We2, be2 = We2s[l], be2s[l]
        dot = lambda x, y: jnp.dot(x, y, preferred_element_type=F32)
        WAs[l] = dot(Wq, Wa1_q)
        WBs[l] = dot(Wk, Wa1_k)
        WVs[l] = dot(Wv, Wm1_v)
        WCs[l] = dot(We2, Wa1_e)
        cCs[l] = ba1s[l] + dot(bq, Wa1_q) + dot(bk, Wa1_k) + dot(be2, Wa1_e)
        WMs[l] = dot(We2, Wm1_e)
        cMs[l] = bm1s[l] + dot(bv, Wm1_v) + dot(be2, Wm1_e)
        w2 = wa2s[l].reshape(4, 16)
        b2 = jnp.full((4, 16), ba2s[l][0, 0], F32)
        cons[l] = jnp.concatenate([w2, b2], axis=0)
    G[...] = jnp.dot(gf[...], Wge[...], preferred_element_type=F32) + bge[...]


def _run_prep(params, graph_features):
    blocks = params['blocks']
    st = lambda key: jnp.stack([b[key][0] for b in blocks])
    stb = lambda key: jnp.stack([b[key][1].reshape(1, -1) for b in blocks])
    Wge, bge = params['graph_enc']
    outs = pl.pallas_call(
        _prep_body,
        out_shape=[
            jax.ShapeDtypeStruct((_NL, 64, 64), F32),
            jax.ShapeDtypeStruct((_NL, 64, 64), F32),
            jax.ShapeDtypeStruct((_NL, 64, 64), F32),
            jax.ShapeDtypeStruct((_NL, 64, 64), F32),
            jax.ShapeDtypeStruct((_NL, 1, 64), F32),
            jax.ShapeDtypeStruct((_NL, 64, 64), F32),
            jax.ShapeDtypeStruct((_NL, 1, 64), F32),
            jax.ShapeDtypeStruct((_NL, 8, 16), F32),
            jax.ShapeDtypeStruct((_NG, 64), F32),
        ],
    )(st('q'), stb('q'), st('k'), stb('k'), st('v'), stb('v'),
      st('att1'), stb('att1'), st('att2'), stb('att2'),
      st('edge2'), stb('edge2'), st('msg1'), stb('msg1'),
      graph_features, Wge, bge.reshape(1, -1))
    return outs


def _enc_body(x, bcol, Wne, bne, G, h):
    oh = (bcol[...] == lax.broadcasted_iota(jnp.int32, (x.shape[0], _NG), 1))
    h[...] = (jnp.dot(x[...], Wne[...], preferred_element_type=F32) + bne[...]
              + jnp.dot(oh.astype(F32), G[...], preferred_element_type=F32))


def _run_enc(x, batch, Wne, bne, G):
    nb = 50
    bs = _N // nb
    return pl.pallas_call(
        _enc_body,
        grid=(nb,),
        in_specs=[
            pl.BlockSpec((bs, 128), lambda i: (i, 0)),
            pl.BlockSpec((bs, 1), lambda i: (i, 0)),
            pl.BlockSpec((128, 64), lambda i: (0, 0)),
            pl.BlockSpec((1, 64), lambda i: (0, 0)),
            pl.BlockSpec((_NG, 64), lambda i: (0, 0)),
        ],
        out_specs=pl.BlockSpec((bs, 64), lambda i: (i, 0)),
        out_shape=jax.ShapeDtypeStruct((_N, 64), F32),
    )(x, batch.reshape(-1, 1), Wne, bne.reshape(1, -1), G)


def _edgepre_body(a, w1s, b1s, WCs, cCs, WMs, cMs, C0, C1, C2, M0, M1, M2):
    Couts = (C0, C1, C2)
    Mouts = (M0, M1, M2)
    for l in range(_NL):
        r = jnp.maximum(a[...] * w1s[l] + b1s[l], 0.0)
        Couts[l][...] = (jnp.dot(r, WCs[l], preferred_element_type=F32)
                         + cCs[l])
        M = jnp.dot(r, WMs[l], preferred_element_type=F32) + cMs[l]
        for qq in range(4):
            Mouts[l][qq] = M[:, qq * 16:(qq + 1) * 16]


def _run_edgepre(edge_attr, w1s, b1s, WCs, cCs, WMs, cMs):
    nb = 400
    bs = _E // nb
    cshape = jax.ShapeDtypeStruct((_E, 64), F32)
    mshape = jax.ShapeDtypeStruct((4, _E, 16), F32)
    full = lambda s: pl.BlockSpec(s, lambda i: tuple(0 for _ in s))
    return pl.pallas_call(
        _edgepre_body,
        grid=(nb,),
        in_specs=[
            pl.BlockSpec((bs, 1), lambda i: (i, 0)),
            full((_NL, 1, 64)), full((_NL, 1, 64)),
            full((_NL, 64, 64)), full((_NL, 1, 64)),
            full((_NL, 64, 64)), full((_NL, 1, 64)),
        ],
        out_specs=[
            pl.BlockSpec((bs, 64), lambda i: (i, 0)),
            pl.BlockSpec((bs, 64), lambda i: (i, 0)),
            pl.BlockSpec((bs, 64), lambda i: (i, 0)),
            pl.BlockSpec((4, bs, 16), lambda i: (0, i, 0)),
            pl.BlockSpec((4, bs, 16), lambda i: (0, i, 0)),
            pl.BlockSpec((4, bs, 16), lambda i: (0, i, 0)),
        ],
        out_shape=[cshape, cshape, cshape, mshape, mshape, mshape],
    )(edge_attr, w1s, b1s, WCs, cCs, WMs, cMs)


def _nodepre_body(h, WA, WB, WV, A, B, V):
    hv = h[...]
    dot = lambda x, y: jnp.dot(x, y, preferred_element_type=F32)
    A[...] = dot(hv, WA[...])
    B[...] = dot(hv, WB[...])
    Vf = dot(hv, WV[...])
    for qq in range(4):
        V[qq] = Vf[:, qq * 16:(qq + 1) * 16]


def _run_nodepre(h, WA, WB, WV):
    nb = 50
    bs = _N // nb
    return pl.pallas_call(
        _nodepre_body,
        grid=(nb,),
        in_specs=[
            pl.BlockSpec((bs, 64), lambda i: (i, 0)),
            pl.BlockSpec((64, 64), lambda i: (0, 0)),
            pl.BlockSpec((64, 64), lambda i: (0, 0)),
            pl.BlockSpec((64, 64), lambda i: (0, 0)),
        ],
        out_specs=[
            pl.BlockSpec((bs, 64), lambda i: (i, 0)),
            pl.BlockSpec((bs, 64), lambda i: (i, 0)),
            pl.BlockSpec((4, bs, 16), lambda i: (0, i, 0)),
        ],
        out_shape=[
            jax.ShapeDtypeStruct((_N, 64), F32),
            jax.ShapeDtypeStruct((_N, 64), F32),
            jax.ShapeDtypeStruct((4, _N, 16), F32),
        ],
    )(h, WA, WB, WV)


def _sc_scores_body(col_hbm, row_hbm, A_hbm, B_hbm, C_hbm, cons_hbm,
                    s_hbm, mparts_hbm,
                    m_tbl, cidx, ridx, abuf, bbuf, cbuf, sbuf, consts,
                    isem, gsem):
    core = lax.axis_index("c")
    sid = lax.axis_index("s")
    wid = sid * 2 + core

    pltpu.sync_copy(cons_hbm, consts)

    def init_body(i, _):
        m_tbl[pl.ds(i * 16, 16)] = jnp.full((16,), -3e38, F32)
        return 0
    lax.fori_loop(0, _N // 16, init_body, 0)

    w2v = tuple(consts[j, pl.ds(0, 16)] for j in range(4))
    lane = lax.iota(jnp.int32, 16)

    nc = _E // _K1
    ntbase = nc // _NTILE
    nt = ntbase + jnp.where(wid < (nc - ntbase * _NTILE), 1, 0)

    def chunk_body(t, _):
        base = (wid + t * _NTILE) * _K1
        for j in range(2):
            pltpu.async_copy(col_hbm.at[pl.ds(base + j * 128, 128)],
                             cidx.at[j], isem)
            pltpu.async_copy(row_hbm.at[pl.ds(base + j * 128, 128)],
                             ridx.at[j], isem)
        for _i in range(4):
            pltpu.make_async_copy(col_hbm.at[pl.ds(base, 128)],
                                  cidx.at[0], isem).wait()
        for j in range(2):
            pltpu.async_copy(A_hbm.at[cidx.at[j]],
                             abuf.at[pl.ds(j * 128, 128)], gsem)
            pltpu.async_copy(B_hbm.at[ridx.at[j]],
                             bbuf.at[pl.ds(j * 128, 128)], gsem)
        pltpu.async_copy(C_hbm.at[pl.ds(base, _K1)], cbuf, gsem)
        for j in range(2):
            pltpu.make_async_copy(A_hbm.at[cidx.at[j]],
                                  abuf.at[pl.ds(j * 128, 128)], gsem).wait()
            pltpu.make_async_copy(B_hbm.at[ridx.at[j]],
                                  bbuf.at[pl.ds(j * 128, 128)], gsem).wait()
        pltpu.make_async_copy(C_hbm.at[pl.ds(base, _K1)], cbuf, gsem).wait()

        def group_body(g, _):
            s16 = consts[4, pl.ds(0, 16)]
            for k in range(16):
                i = g * 16 + k
                acc = jnp.zeros((16,), F32)
                for j in range(4):
                    u = (abuf[i, pl.ds(j * 16, 16)]
                         + bbuf[i, pl.ds(j * 16, 16)]
                         + cbuf[i, pl.ds(j * 16, 16)])
                    acc = acc + jnp.maximum(u, 0.0) * w2v[j]
                tot = plsc.cumsum(acc)[15]
                s16 = jnp.where(lane == k, tot, s16)
            sbuf[pl.ds(g * 16, 16)] = s16
            c16 = cidx[g // 8, pl.ds((g % 8) * 16, 16)]
            old = plsc.load_gather(m_tbl, [c16])
            plsc.store_scatter(m_tbl, [c16], jnp.maximum(old, s16))
            return 0
        lax.fori_loop(0, _K1 // 16, group_body, 0)

        pltpu.sync_copy(sbuf, s_hbm.at[pl.ds(base, _K1)])
        return 0
    lax.fori_loop(0, nt, chunk_body, 0)

    pltpu.sync_copy(m_tbl, mparts_hbm.at[wid])


def _run_sc_scores(col, row, A, B, C, cons):
    mesh = plsc.VectorSubcoreMesh(core_axis_name="c", subcore_axis_name="s")
    kern = functools.partial(
        pl.kernel,
        mesh=mesh,
        compiler_params=pltpu.CompilerParams(needs_layout_passes=False, use_tc_tiling_on_sc=False),
        out_type=[
            jax.ShapeDtypeStruct((_E,), F32),
            jax.ShapeDtypeStruct((_NTILE, _N), F32),
        ],
        scratch_types=[
            pltpu.VMEM((_N,), F32),
            pltpu.VMEM((2, 128), jnp.int32),
            pltpu.VMEM((2, 128), jnp.int32),
            pltpu.VMEM((_K1, 64), F32),
            pltpu.VMEM((_K1, 64), F32),
            pltpu.VMEM((_K1, 64), F32),
            pltpu.VMEM((_K1,), F32),
            pltpu.VMEM((8, 16), F32),
            pltpu.SemaphoreType.DMA,
            pltpu.SemaphoreType.DMA,
        ],
    )(_sc_scores_body)
    return kern(col, row, A, B, C, cons)


_MSZ_A = 1568
_MSZ_B = 1392


def _sc_mmerge_body(mparts_hbm, m_hbm, mbuf, sem):
    core = lax.axis_index("c")
    sid = lax.axis_index("s")
    wid = sid * 2 + core

    def merge_range(start, size):
        pltpu.sync_copy(mparts_hbm.at[pl.ds(0, _NTILE), pl.ds(start, size)],
                        mbuf.at[pl.ds(0, _NTILE), pl.ds(0, size)])

        def red_body(i, _):
            acc = mbuf[0, pl.ds(i * 16, 16)]
            for k in range(1, _NTILE):
                acc = jnp.maximum(acc, mbuf[k, pl.ds(i * 16, 16)])
            mbuf[0, pl.ds(i * 16, 16)] = acc
            return 0
        lax.fori_loop(0, size // 16, red_body, 0)
        pltpu.sync_copy(mbuf.at[0, pl.ds(0, size)],
                        m_hbm.at[pl.ds(start, size)])

    @pl.when(wid < _NTILE - 1)
    def _():
        merge_range(wid * _MSZ_A, _MSZ_A)

    @pl.when(wid == _NTILE - 1)
    def _():
        merge_range((_NTILE - 1) * _MSZ_A, _MSZ_B)


def _run_sc_mmerge(mparts):
    mesh = plsc.VectorSubcoreMesh(core_axis_name="c", subcore_axis_name="s")
    kern = functools.partial(
        pl.kernel,
        mesh=mesh,
        compiler_params=pltpu.CompilerParams(needs_layout_passes=False, use_tc_tiling_on_sc=False),
        out_type=[jax.ShapeDtypeStruct((_N,), F32)],
        scratch_types=[
            pltpu.VMEM((_NTILE, _MSZ_A), F32),
            pltpu.SemaphoreType.DMA,
        ],
    )(_sc_mmerge_body)
    return kern(mparts)


def _sc_agg_body(col_hbm, row_hbm, s_all_hbm, m_hbm, V_hbm, M_hbm,
                 zeros16_hbm,
                 T_hbm, S_hbm,
                 m_tbl, cidx, ridx, sbuf, vbuf, mbuf, Tsh, isem, gsem):
    core = lax.axis_index("c")
    sid = lax.axis_index("s")
    lane = lax.iota(jnp.int32, 16)

    pltpu.sync_copy(m_hbm, m_tbl)

    nc = _E // _K2
    ntb = nc // 16
    nt = ntb + jnp.where(sid < (nc - ntb * 16), 1, 0)

    def zero_tsh():
        @pl.when(sid < 15)
        def _():
            pltpu.sync_copy(zeros16_hbm, Tsh.at[pl.ds(sid * _ROWS_A, _ROWS_A)])

        @pl.when(sid == 15)
        def _():
            pltpu.sync_copy(zeros16_hbm.at[pl.ds(0, _ROWS_B)],
                            Tsh.at[pl.ds(15 * _ROWS_A, _ROWS_B)])

    def copy_tsh_out(dst):
        @pl.when(sid < 15)
        def _():
            r0 = sid * _ROWS_A
            pltpu.sync_copy(Tsh.at[pl.ds(r0, _ROWS_A)],
                            dst.at[pl.ds(r0, _ROWS_A)])

        @pl.when(sid == 15)
        def _():
            r0 = 15 * _ROWS_A
            pltpu.sync_copy(Tsh.at[pl.ds(r0, _ROWS_B)],
                            dst.at[pl.ds(r0, _ROWS_B)])

    def load_idx(base, want_row):
        for j in range(5):
            pltpu.async_copy(col_hbm.at[pl.ds(base + j * 128, 128)],
                             cidx.at[j], isem)
            if want_row:
                pltpu.async_copy(row_hbm.at[pl.ds(base + j * 128, 128)],
                                 ridx.at[j], isem)
        pltpu.async_copy(s_all_hbm.at[pl.ds(base, _K2)], sbuf, isem)
        n = 10 if want_row else 5
        for _i in range(n):
            pltpu.make_async_copy(col_hbm.at[pl.ds(base, 128)],
                                  cidx.at[0], isem).wait()
        pltpu.make_async_copy(s_all_hbm.at[pl.ds(base, _K2)], sbuf,
                              isem).wait()

    def e_of_group(g):
        c16 = cidx[g // 8, pl.ds((g % 8) * 16, 16)]
        s16 = sbuf[pl.ds(g * 16, 16)]
        mg = plsc.load_gather(m_tbl, [c16])
        return jnp.exp(s16 - mg)

    for ph in range(2):
        q = core * 2 + ph
        zero_tsh()
        plsc.subcore_barrier()

        def chunk_body(t, _):
            base = (sid + t * 16) * _K2
            load_idx(base, True)
            for j in range(5):
                pltpu.async_copy(V_hbm.at[q].at[ridx.at[j]],
                                 vbuf.at[pl.ds(j * 128, 128)], gsem)
            pltpu.async_copy(M_hbm.at[q, pl.ds(base, _K2)], mbuf, gsem)
            for j in range(5):
                pltpu.make_async_copy(V_hbm.at[q].at[ridx.at[j]],
                                      vbuf.at[pl.ds(j * 128, 128)],
                                      gsem).wait()
            pltpu.make_async_copy(M_hbm.at[q, pl.ds(base, _K2)], mbuf,
                                  gsem).wait()

            def group_body(g, _):
                e16 = e_of_group(g)
                for k in range(16):
                    i = g * 16 + k
                    ev = jnp.full((16,), e16[k], F32)
                    u = vbuf[i, pl.ds(0, 16)] + mbuf[i, pl.ds(0, 16)]
                    vbuf[i, pl.ds(0, 16)] = ev * (u / (1.0 + jnp.exp(-u)))
                return 0
            lax.fori_loop(0, _K2 // 16, group_body, 0)

            for j in range(5):
                pltpu.sync_copy(vbuf.at[pl.ds(j * 128, 128)],
                                Tsh.at[cidx.at[j]], add=True)
            return 0
        lax.fori_loop(0, nt, chunk_body, 0)

        plsc.subcore_barrier()
        copy_tsh_out(T_hbm.at[q])
        plsc.subcore_barrier()

    def zv_body(i, _):
        vbuf[i, pl.ds(0, 16)] = jnp.zeros((16,), F32)
        return 0
    lax.fori_loop(0, _K2, zv_body, 0)
    zero_tsh()
    plsc.subcore_barrier()

    wid = core * 16 + sid
    ntb3 = nc // _NTILE
    nt3 = ntb3 + jnp.where(wid < (nc - ntb3 * _NTILE), 1, 0)

    def s_chunk(t, _):
        base = (wid + t * _NTILE) * _K2
        load_idx(base, False)

        def sg_body(g, _):
            e16 = e_of_group(g)
            plsc.store_scatter(vbuf, [lane + g * 16,
                                      jnp.zeros((16,), jnp.int32)], e16)
            return 0
        lax.fori_loop(0, _K2 // 16, sg_body, 0)

        for j in range(5):
            pltpu.sync_copy(vbuf.at[pl.ds(j * 128, 128)],
                            Tsh.at[cidx.at[j]], add=True)
        return 0
    lax.fori_loop(0, nt3, s_chunk, 0)

    plsc.subcore_barrier()
    copy_tsh_out(S_hbm.at[core])


def _run_sc_agg(col, row, s_all, m, V, M, zeros16):
    mesh = plsc.VectorSubcoreMesh(core_axis_name="c", subcore_axis_name="s")
    kern = functools.partial(
        pl.kernel,
        mesh=mesh,
        compiler_params=pltpu.CompilerParams(needs_layout_passes=False, use_tc_tiling_on_sc=False),
        out_type=[
            jax.ShapeDtypeStruct((4, _N, 16), F32),
            jax.ShapeDtypeStruct((2, _N, 16), F32),
        ],
        scratch_types=[
            pltpu.VMEM((_N,), F32),
            pltpu.VMEM((5, 128), jnp.int32),
            pltpu.VMEM((5, 128), jnp.int32),
            pltpu.VMEM((_K2,), F32),
            pltpu.VMEM((_K2, 16), F32),
            pltpu.VMEM((_K2, 16), F32),
            pltpu.VMEM_SHARED((_N, 16), F32),
            pltpu.SemaphoreType.DMA,
            pltpu.SemaphoreType.DMA,
        ],
    )(_sc_agg_body)
    return kern(col, row, s_all, m, V, M, zeros16)


def _post_body(T, Sq, h, Wm2, bm2, hnew):
    S = Sq[0][:, 0:1] + Sq[1][:, 0:1]
    Tc = jnp.concatenate([T[0], T[1], T[2], T[3]], axis=-1)
    w = Tc / (S + 1e-16)
    agg = (jnp.dot(w, Wm2[...], preferred_element_type=F32)
           + jnp.where(S > 0.0, 1.0, 0.0) * bm2[...])
    hnew[...] = h[...] + agg


def _run_post(T, Sq, h, Wm2, bm2):
    nb = 50
    bs = _N // nb
    return pl.pallas_call(
        _post_body,
        grid=(nb,),
        in_specs=[
            pl.BlockSpec((4, bs, 16), lambda i: (0, i, 0)),
            pl.BlockSpec((2, bs, 16), lambda i: (0, i, 0)),
            pl.BlockSpec((bs, 64), lambda i: (i, 0)),
            pl.BlockSpec((64, 64), lambda i: (0, 0)),
            pl.BlockSpec((1, 64), lambda i: (0, 0)),
        ],
        out_specs=pl.BlockSpec((bs, 64), lambda i: (i, 0)),
        out_shape=jax.ShapeDtypeStruct((_N, 64), F32),
    )(T, Sq, h, Wm2, bm2.reshape(1, -1))


def _pool_body(h, bcol, acc_out, acc):
    i = pl.program_id(0)

    @pl.when(i == 0)
    def _():
        acc[...] = jnp.zeros_like(acc)

    oh = (bcol[...] == lax.broadcasted_iota(jnp.int32, (h.shape[0], _NG), 1))
    ohf = oh.astype(F32)
    hh = jnp.concatenate([h[...], jnp.ones_like(h[...])], axis=-1)
    acc[...] += lax.dot_general(ohf, hh, (((0,), (0,)), ((), ())),
                                preferred_element_type=F32)

    @pl.when(i == pl.num_programs(0) - 1)
    def _():
        acc_out[...] = acc[...]


def _run_pool(h, batch):
    nb = 50
    bs = _N // nb
    return pl.pallas_call(
        _pool_body,
        grid=(nb,),
        in_specs=[
            pl.BlockSpec((bs, 64), lambda i: (i, 0)),
            pl.BlockSpec((bs, 1), lambda i: (i, 0)),
        ],
        out_specs=pl.BlockSpec((_NG, 128), lambda i: (0, 0)),
        out_shape=jax.ShapeDtypeStruct((_NG, 128), F32),
        scratch_shapes=[pltpu.VMEM((_NG, 128), F32)],
    )(h, batch.reshape(-1, 1))


def _heads_body(acc, *refs):
    hsum = acc[...][:, 0:64]
    cnt = acc[...][:, 64:65]
    hg = hsum / jnp.maximum(cnt, 1.0)
    nw = [3, 4, 3, 3, 3]
    order = [0, 1, 1, 0, 0]
    pos = 0
    preds = []
    ins = refs[:-1]
    out = refs[-1]
    for hi in range(5):
        z = hg
        depth = 4 if order[hi] else 3
        for d in range(depth):
            W = ins[pos][...]
            b = ins[pos + 1][...]
            pos += 2
            z = jnp.dot(z, W, preferred_element_type=F32) + b
            if d < depth - 1:
                z = _silu(z)
        preds.append(z)
    out[...] = jnp.concatenate(preds, axis=-1)


def _run_heads(acc, heads):
    order = ['neff', 'Aeff', 'NL', 'Disp', 'GVD']
    args = [acc]
    for name in order:
        for (W, b) in heads[name]:
            args.append(W)
            args.append(b.reshape(1, -1))
    return pl.pallas_call(
        _heads_body,
        out_shape=jax.ShapeDtypeStruct((_NG, 5), F32),
    )(*args)


def kernel(x, pos, graph_features, batch, edge_index, edge_attr, params):
    del pos
    (WAs, WBs, WVs, WCs, cCs, WMs, cMs, cons, G) = _run_prep(
        params, graph_features)
    Wne, bne = params['node_enc']
    h = _run_enc(x, batch, Wne, bne, G)

    blocks = params['blocks']
    w1s = jnp.stack([b['edge1'][0] for b in blocks])
    b1s = jnp.stack([b['edge1'][1].reshape(1, -1) for b in blocks])
    C0, C1, C2, M0, M1, M2 = _run_edgepre(edge_attr, w1s, b1s,
                                          WCs, cCs, WMs, cMs)
    Cs = (C0, C1, C2)
    Ms = (M0, M1, M2)

    col = edge_index[1]
    row = edge_index[0]
    zeros16 = jnp.zeros((_ROWS_A, 16), F32)

    for l in range(_NL):
        A, B, V = _run_nodepre(h, WAs[l], WBs[l], WVs[l])
        s_all, mparts = _run_sc_scores(col, row, A, B, Cs[l], cons[l])
        m, = _run_sc_mmerge(mparts)
        T, Sq = _run_sc_agg(col, row, s_all, m, V, Ms[l], zeros16)
        Wm2, bm2 = blocks[l]['msg2']
        h = _run_post(T, Sq, h, Wm2, bm2)

    acc = _run_pool(h, batch)
    return _run_heads(acc, params['heads'])
